# baseline (speedup 1.0000x reference)
import functools

import jax
import jax.numpy as jnp
from jax import lax
from jax.experimental import pallas as pl
from jax.experimental.pallas import tpu as pltpu
from jax.experimental.pallas import tpu_sc as plsc

B = 4096
F = 64
FD = 32
K = 2048
NC, NS = 2, 16
NW = NC * NS
BPW = B // NW
BLK = 512
GRID = B // BLK


@functools.cache
def _make_sc_bi_gather():
    mesh = plsc.VectorSubcoreMesh(core_axis_name="c", subcore_axis_name="s",
                                  num_cores=NC, num_subcores=NS)

    @functools.partial(
        pl.kernel,
        out_type=jax.ShapeDtypeStruct((B,), jnp.float32),
        mesh=mesh,
        scratch_types=[
            pltpu.VMEM((BPW,), jnp.int32),
            pltpu.VMEM((BPW,), jnp.float32),
            pltpu.SemaphoreType.DMA,
        ],
        compiler_params=pltpu.CompilerParams(use_tc_tiling_on_sc=False),
    )
    def _sc_bi(item_hbm, bi_hbm, bi_out, iidx_v, bi_v, sem):
        wid = lax.axis_index("s") * NC + lax.axis_index("c")
        base = wid * BPW
        pltpu.sync_copy(item_hbm.at[pl.ds(base, BPW)], iidx_v)
        pltpu.async_copy(bi_hbm.at[iidx_v], bi_v, sem).wait()
        pltpu.sync_copy(bi_v, bi_out.at[pl.ds(base, BPW)])

    return _sc_bi


def _mega_body(user_ref, item_ref, beta_ref, feat_ref, eb_ref,
               gu_hbm, gi_hbm, tu_hbm,
               gu_out, gi_out, tu_out, xui_ref,
               gu_v, gi_v, tu_v, sem):
    i = pl.program_id(0)
    base = i * BLK

    def body(r, carry):
        u = user_ref[base + r]
        it = item_ref[base + r]
        pltpu.async_copy(gu_hbm.at[u], gu_v.at[r], sem)
        pltpu.async_copy(tu_hbm.at[u], tu_v.at[r], sem)
        pltpu.async_copy(gi_hbm.at[it], gi_v.at[r], sem)
        return carry

    lax.fori_loop(0, BLK, body, 0)
    feb = jnp.dot(feat_ref[...], eb_ref[...],
                  preferred_element_type=jnp.float32)
    pltpu.make_async_copy(gu_hbm.at[pl.ds(0, BLK)], gu_v, sem).wait()
    pltpu.make_async_copy(tu_hbm.at[pl.ds(0, BLK)], tu_v, sem).wait()
    pltpu.make_async_copy(gi_hbm.at[pl.ds(0, BLK)], gi_v, sem).wait()
    gu = gu_v[...]
    gi = gi_v[...]
    tu = tu_v[...]
    xui_ref[0, 0, :] = (beta_ref[0, 0, :]
                        + jnp.sum(gu * gi, axis=1)
                        + jnp.sum(tu * feb[:, :FD], axis=1)
                        + feb[:, FD])
    gu_out[...] = gu
    gi_out[...] = gi
    tu_out[...] = tu


_tc_mega = pl.pallas_call(
    _mega_body,
    grid=(GRID,),
    in_specs=[
        pl.BlockSpec(memory_space=pltpu.SMEM),
        pl.BlockSpec(memory_space=pltpu.SMEM),
        pl.BlockSpec((1, 1, BLK), lambda i: (i, 0, 0)),
        pl.BlockSpec((BLK, K), lambda i: (i, 0)),
        pl.BlockSpec((K, FD + 1), lambda i: (0, 0)),
        pl.BlockSpec(memory_space=pltpu.HBM),
        pl.BlockSpec(memory_space=pltpu.HBM),
        pl.BlockSpec(memory_space=pltpu.HBM),
    ],
    out_specs=[
        pl.BlockSpec((BLK, F), lambda i: (i, 0)),
        pl.BlockSpec((BLK, F), lambda i: (i, 0)),
        pl.BlockSpec((BLK, FD), lambda i: (i, 0)),
        pl.BlockSpec((1, 1, BLK), lambda i: (i, 0, 0)),
    ],
    out_shape=[
        jax.ShapeDtypeStruct((B, F), jnp.float32),
        jax.ShapeDtypeStruct((B, F), jnp.float32),
        jax.ShapeDtypeStruct((B, FD), jnp.float32),
        jax.ShapeDtypeStruct((GRID, 1, BLK), jnp.float32),
    ],
    scratch_shapes=[
        pltpu.VMEM((BLK, F), jnp.float32),
        pltpu.VMEM((BLK, F), jnp.float32),
        pltpu.VMEM((BLK, FD), jnp.float32),
        pltpu.SemaphoreType.DMA,
    ],
    compiler_params=pltpu.CompilerParams(
        dimension_semantics=("arbitrary",)),
)


def kernel(user, item, feature_i, Bi, Gu, Gi, Bp, Tu, E):
    beta_i = _make_sc_bi_gather()(item, Bi)
    eb = jnp.concatenate([E, Bp], axis=1)
    gamma_u, gamma_i, theta_u, xui = _tc_mega(
        user, item, beta_i.reshape(GRID, 1, BLK), feature_i, eb,
        Gu, Gi, Tu)
    return (xui.reshape(B), gamma_u, gamma_i, feature_i, theta_u, beta_i)
